# baseline (speedup 1.0000x reference)
import functools

import jax
import jax.numpy as jnp
from jax import lax
from jax.experimental import pallas as pl
from jax.experimental.pallas import tpu as pltpu
from jax.experimental.pallas import tpu_sc as plsc

HID = 1024
OUT_W = HID + 1
NC = 2
NS = 16
NW = NC * NS
CHUNK = 32


def _sc_body(tok_per_w, ids_hbm, maskf_hbm, table_hbm, out_hbm,
             idx_v, rows0, rows1, sem0, sem1, semm):
    c = lax.axis_index("c")
    s = lax.axis_index("s")
    wid = s * NC + c
    base = wid * tok_per_w

    pltpu.sync_copy(ids_hbm.at[pl.ds(base, tok_per_w)], idx_v)
    cp_m = pltpu.async_copy(
        maskf_hbm.at[pl.ds(base, tok_per_w)],
        out_hbm.at[pl.ds(base, tok_per_w), pl.ds(HID, 1)], semm)

    nchunk = tok_per_w // CHUNK
    bufs = (rows0, rows1)
    sems = (sem0, sem1)
    cps = [None, None]
    cps[0] = pltpu.async_copy(
        table_hbm.at[idx_v.at[pl.ds(0, CHUNK)]], rows0, sem0)
    for i in range(nchunk):
        cur = i % 2
        nxt = (i + 1) % 2
        if i + 1 < nchunk:
            cps[nxt] = pltpu.async_copy(
                table_hbm.at[idx_v.at[pl.ds((i + 1) * CHUNK, CHUNK)]],
                bufs[nxt], sems[nxt])
        cps[cur].wait()
        pltpu.sync_copy(
            bufs[cur],
            out_hbm.at[pl.ds(base + i * CHUNK, CHUNK), pl.ds(0, HID)])
    cp_m.wait()


@jax.jit
def kernel(input_ids, attention_mask, table):
    b, s = input_ids.shape
    n = b * s
    tok_per_w = n // NW
    ids = input_ids.reshape(n).astype(jnp.int32)
    maskf = attention_mask.reshape(n, 1).astype(jnp.float32)

    mesh = plsc.VectorSubcoreMesh(core_axis_name="c", subcore_axis_name="s")
    emb = pl.kernel(
        functools.partial(_sc_body, tok_per_w),
        out_type=jax.ShapeDtypeStruct((n, OUT_W), jnp.float32),
        mesh=mesh,
        scratch_types=[
            pltpu.VMEM((tok_per_w,), jnp.int32),
            pltpu.VMEM((CHUNK, HID), jnp.float32),
            pltpu.VMEM((CHUNK, HID), jnp.float32),
            pltpu.SemaphoreType.DMA,
            pltpu.SemaphoreType.DMA,
            pltpu.SemaphoreType.DMA,
        ],
    )(ids, maskf, table)
    return emb.reshape(b, s, OUT_W)


# speedup vs baseline: 2.4617x; 2.4617x over previous
import functools

import jax
import jax.numpy as jnp
from jax import lax
from jax.experimental import pallas as pl
from jax.experimental.pallas import tpu as pltpu
from jax.experimental.pallas import tpu_sc as plsc

HID = 1024
OUT_W = HID + 1
NC = 2
NS = 16
NW = NC * NS
CHUNK = 32


def _gather_body(tok_per_w, ids_hbm, table_hbm, out_hbm,
                 idx_v, rows0, rows1, sem0, sem1):
    c = lax.axis_index("c")
    s = lax.axis_index("s")
    wid = s * NC + c
    base = wid * tok_per_w

    pltpu.sync_copy(ids_hbm.at[pl.ds(base, tok_per_w)], idx_v)

    nchunk = tok_per_w // CHUNK
    bufs = (rows0, rows1)
    sems = (sem0, sem1)
    cps = [None, None]
    cps[0] = pltpu.async_copy(
        table_hbm.at[idx_v.at[pl.ds(0, CHUNK)]], rows0, sem0)
    for i in range(nchunk):
        cur = i % 2
        nxt = (i + 1) % 2
        if i + 1 < nchunk:
            cps[nxt] = pltpu.async_copy(
                table_hbm.at[idx_v.at[pl.ds((i + 1) * CHUNK, CHUNK)]],
                bufs[nxt], sems[nxt])
        cps[cur].wait()
        pltpu.sync_copy(
            bufs[cur],
            out_hbm.at[pl.ds(base + i * CHUNK, CHUNK), pl.ds(0, HID)])


def _mask_body(tok_per_w, maskt_ref, emb_ref, out_ref):
    del emb_ref
    i = pl.program_id(0)
    m = maskt_ref[...]
    lane = jax.lax.broadcasted_iota(jnp.int32, m.shape, 1)
    col = jnp.sum(jnp.where(lane == i, m, 0.0), axis=1, keepdims=True)
    out_ref[...] = jnp.broadcast_to(col, (tok_per_w, 128))


@jax.jit
def kernel(input_ids, attention_mask, table):
    b, s = input_ids.shape
    n = b * s
    tok_per_w = n // NW
    ids = input_ids.reshape(n).astype(jnp.int32)

    mesh = plsc.VectorSubcoreMesh(core_axis_name="c", subcore_axis_name="s")
    emb = pl.kernel(
        functools.partial(_gather_body, tok_per_w),
        out_type=jax.ShapeDtypeStruct((n, OUT_W), jnp.float32),
        mesh=mesh,
        scratch_types=[
            pltpu.VMEM((tok_per_w,), jnp.int32),
            pltpu.VMEM((CHUNK, HID), jnp.float32),
            pltpu.VMEM((CHUNK, HID), jnp.float32),
            pltpu.SemaphoreType.DMA,
            pltpu.SemaphoreType.DMA,
        ],
    )(ids, table)

    maskt = attention_mask.reshape(NW, tok_per_w).T.astype(jnp.float32)
    out = pl.pallas_call(
        functools.partial(_mask_body, tok_per_w),
        grid=(NW,),
        in_specs=[
            pl.BlockSpec((tok_per_w, NW), lambda i: (0, 0)),
            pl.BlockSpec(memory_space=pl.ANY),
        ],
        out_specs=pl.BlockSpec((tok_per_w, 128), lambda i: (i, HID // 128)),
        out_shape=jax.ShapeDtypeStruct((n, OUT_W), jnp.float32),
        input_output_aliases={1: 0},
    )(maskt, emb)
    return out.reshape(b, s, OUT_W)
